# baseline (speedup 1.0000x reference)
import functools

import jax
import jax.numpy as jnp
import numpy as np
from jax import lax
from jax.experimental import pallas as pl
from jax.experimental.pallas import tpu as pltpu
from jax.experimental.pallas import tpu_sc as plsc

NA = 9
FH = 64
FW = 64
B = 4
NG = 50
A = FH * FW * NA
AR = A // 128
GP = 64
R1 = 16
NJ = AR // R1

_INTERPRET = False


def _anchors_np():
    base_size = 16
    anchors = []
    cx = base_size / 2.0
    cy = base_size / 2.0
    for r in (0.5, 1.0, 2.0):
        for s in (8, 16, 32):
            area = float(base_size * s) ** 2
            w = np.sqrt(area / r)
            h = w * r
            anchors.append([cx - 0.5 * w, cy - 0.5 * h, cx + 0.5 * w, cy + 0.5 * h])
    base = np.array(anchors, dtype=np.float32)
    shift_x = np.arange(FW, dtype=np.float32) * 16.0
    shift_y = np.arange(FH, dtype=np.float32) * 16.0
    sx, sy = np.meshgrid(shift_x, shift_y, indexing="ij")
    shifts = np.stack([sx, sy, sx, sy], axis=-1).reshape(-1, 4).astype(np.float32)
    return (base[None, :, :] + shifts[:, None, :]).reshape(-1, 4).astype(np.float32)


_ANCH = _anchors_np()
_AX = [np.ascontiguousarray(_ANCH[:, c].reshape(AR, 128)) for c in range(4)]


def _s1_body(ax1r, ay1r, ax2r, ay2r, g0r, g1r, g2r, g3r, vmr,
             mir, m0r, m1r, m2r, m3r):
    ax1 = ax1r[...][None]
    ay1 = ay1r[...][None]
    ax2 = ax2r[...][None]
    ay2 = ay2r[...][None]
    gx1 = g0r[0]
    gy1 = g1r[0]
    gx2 = g2r[0]
    gy2 = g3r[0]
    vm = vmr[0]
    x1 = jnp.maximum(ax1, gx1)
    y1 = jnp.maximum(ay1, gy1)
    x2 = jnp.minimum(ax2, gx2)
    y2 = jnp.minimum(ay2, gy2)
    inter = jnp.maximum(0.0, x2 - x1) * jnp.maximum(0.0, y2 - y1)
    a1 = (ax2 - ax1) * (ay2 - ay1)
    a2 = (gx2 - gx1) * (gy2 - gy1)
    iou = inter / (a1 + a2 - inter + 1e-8)
    iou = iou * vm + (vm - 1.0)
    mx = jnp.max(iou, axis=0)
    it = jax.lax.broadcasted_iota(jnp.int32, (GP, R1, 128), 0)
    am = jnp.min(jnp.where(iou >= mx[None], it, GP), axis=0)
    oh = (it == am[None]).astype(jnp.float32)
    mir[0] = mx
    m0r[0] = jnp.sum(oh * gx1, axis=0)
    m1r[0] = jnp.sum(oh * gy1, axis=0)
    m2r[0] = jnp.sum(oh * gx2, axis=0)
    m3r[0] = jnp.sum(oh * gy2, axis=0)


def _stage1(g, vm):
    anch_spec = pl.BlockSpec((R1, 128), lambda b, j: (j, 0))
    gt_spec = pl.BlockSpec((1, GP, 1, 1), lambda b, j: (b, 0, 0, 0))
    out_spec = pl.BlockSpec((1, R1, 128), lambda b, j: (b, j, 0))
    shp = jax.ShapeDtypeStruct((B, AR, 128), jnp.float32)
    return pl.pallas_call(
        _s1_body,
        grid=(B, NJ),
        in_specs=[anch_spec] * 4 + [gt_spec] * 5,
        out_specs=[out_spec] * 5,
        out_shape=[shp] * 5,
        interpret=_INTERPRET,
    )(*_AX, *g, vm)


NW = 32
APW = A // NW
NCH = APW // 16
Q = 4
_AXF = [np.ascontiguousarray(_ANCH[:, c]) for c in range(4)]


_AXPACK = np.ascontiguousarray(
    _ANCH.T.reshape(4, NW, APW).transpose(1, 0, 2).reshape(-1))


def _sc_match(gt_pack, nv):
    mesh = plsc.VectorSubcoreMesh(core_axis_name="c", subcore_axis_name="s")
    shp = jax.ShapeDtypeStruct((NW * B * 5 * APW,), jnp.float32)

    @functools.partial(
        pl.kernel,
        mesh=mesh,
        out_type=shp,
        scratch_types=[pltpu.VMEM((4 * APW,), jnp.float32),
                       pltpu.VMEM((B * 5 * GP,), jnp.float32),
                       pltpu.VMEM((16,), jnp.int32),
                       pltpu.VMEM((B * 5 * APW,), jnp.float32)],
        compiler_params=pltpu.CompilerParams(needs_layout_passes=False),
    )
    def k(axh, gth, nvh, outh, axv, gtv, nvv, outv):
        wid = lax.axis_index("s") * 2 + lax.axis_index("c")
        pltpu.sync_copy(axh.at[pl.ds(wid * 4 * APW, 4 * APW)], axv)
        pltpu.sync_copy(gth, gtv)
        pltpu.sync_copy(nvh, nvv)
        for b in range(B):
            nvb = nvv[...][b]
            gof = b * 5 * GP

            def quad(q, _):
                off = [q * (16 * Q) + i * 16 for i in range(Q)]
                ax1 = [axv[pl.ds(0 * APW + o, 16)] for o in off]
                ay1 = [axv[pl.ds(1 * APW + o, 16)] for o in off]
                ax2 = [axv[pl.ds(2 * APW + o, 16)] for o in off]
                ay2 = [axv[pl.ds(3 * APW + o, 16)] for o in off]
                a1 = [(ax2[i] - ax1[i]) * (ay2[i] - ay1[i]) for i in range(Q)]

                def gt_step(g, carry):
                    rmax, ridx = carry
                    gv = jnp.full((16,), g, jnp.int32)
                    gx1 = plsc.load_gather(gtv, [gv + (gof + 0 * GP)])
                    gy1 = plsc.load_gather(gtv, [gv + (gof + 1 * GP)])
                    gx2 = plsc.load_gather(gtv, [gv + (gof + 2 * GP)])
                    gy2 = plsc.load_gather(gtv, [gv + (gof + 3 * GP)])
                    ga = plsc.load_gather(gtv, [gv + (gof + 4 * GP)])
                    nmax, nidx = [], []
                    for i in range(Q):
                        iw = jnp.maximum(
                            0.0, jnp.minimum(ax2[i], gx2) - jnp.maximum(ax1[i], gx1))
                        ih = jnp.maximum(
                            0.0, jnp.minimum(ay2[i], gy2) - jnp.maximum(ay1[i], gy1))
                        inter = iw * ih
                        iou = inter / (a1[i] + ga - inter + 1e-8)
                        upd = iou > rmax[i]
                        nmax.append(jnp.where(upd, iou, rmax[i]))
                        nidx.append(jnp.where(upd, gv, ridx[i]))
                    return tuple(nmax), tuple(nidx)

                rmax = tuple(jnp.full((16,), -1.0, jnp.float32) for _ in range(Q))
                ridx = tuple(jnp.zeros((16,), jnp.int32) for _ in range(Q))
                rmax, ridx = lax.fori_loop(0, nvb, gt_step, (rmax, ridx))
                oof = b * 5 * APW
                for i in range(Q):
                    outv[pl.ds(oof + 0 * APW + off[i], 16)] = rmax[i]
                    outv[pl.ds(oof + 1 * APW + off[i], 16)] = plsc.load_gather(
                        gtv, [ridx[i] + (gof + 0 * GP)])
                    outv[pl.ds(oof + 2 * APW + off[i], 16)] = plsc.load_gather(
                        gtv, [ridx[i] + (gof + 1 * GP)])
                    outv[pl.ds(oof + 3 * APW + off[i], 16)] = plsc.load_gather(
                        gtv, [ridx[i] + (gof + 2 * GP)])
                    outv[pl.ds(oof + 4 * APW + off[i], 16)] = plsc.load_gather(
                        gtv, [ridx[i] + (gof + 3 * GP)])
                return 0

            lax.fori_loop(0, 1, quad, 0)
        pltpu.sync_copy(outv, outh.at[pl.ds(wid * B * 5 * APW, B * 5 * APW)])

    return k(_AXPACK, gt_pack, nv)


def _smooth_l1(d):
    ad = jnp.abs(d)
    return jnp.where(ad < 1.0, 0.5 * d * d, ad - 0.5)


def _s2_body(mir, m0r, m1r, m2r, m3r, l0r, l1r, p0r, p1r, p2r, p3r,
             ax1r, ay1r, ax2r, ay2r, labr, tgtr, cer, slr, cntr):
    b = pl.program_id(0)
    mx = mir[0]
    lab = mx >= 0.7
    labf = lab.astype(jnp.float32)
    labr[0] = lab.astype(jnp.int32)
    ax1 = ax1r[...]
    ay1 = ay1r[...]
    ax2 = ax2r[...]
    ay2 = ay2r[...]
    bw = ax2 - ax1 + 1.0
    bh = ay2 - ay1 + 1.0
    bcx = ax1 + 0.5 * bw
    bcy = ay1 + 0.5 * bh
    m0 = m0r[0]
    m1 = m1r[0]
    m2 = m2r[0]
    m3 = m3r[0]
    gw = m2 - m0 + 1.0
    gh = m3 - m1 + 1.0
    gcx = m0 + 0.5 * gw
    gcy = m1 + 0.5 * gh
    t0 = (gcx - bcx) / bw
    t1 = (gcy - bcy) / bh
    t2 = jnp.log(gw / bw)
    t3 = jnp.log(gh / bh)
    tgtr[0, 0] = t0
    tgtr[0, 1] = t1
    tgtr[0, 2] = t2
    tgtr[0, 3] = t3
    l0 = l0r[0]
    l1 = l1r[0]
    mm = jnp.maximum(l0, l1)
    lse = mm + jnp.log(jnp.exp(l0 - mm) + jnp.exp(l1 - mm))
    ce_b = jnp.sum(lse - jnp.where(lab, l1, l0), keepdims=True)
    sl = (_smooth_l1(p0r[0] - t0) + _smooth_l1(p1r[0] - t1)
          + _smooth_l1(p2r[0] - t2) + _smooth_l1(p3r[0] - t3))
    sl_b = jnp.sum(sl * labf, keepdims=True)
    cnt_b = jnp.sum(labf, keepdims=True)

    @pl.when(b == 0)
    def _():
        cer[...] = jnp.zeros((1, 1), jnp.float32)
        slr[...] = jnp.zeros((1, 1), jnp.float32)
        cntr[...] = jnp.zeros((1, 1), jnp.float32)

    cer[...] += ce_b
    slr[...] += sl_b
    cntr[...] += cnt_b


def _stage2(mi, m, l0, l1, p):
    big = pl.BlockSpec((1, AR, 128), lambda b: (b, 0, 0))
    anch_spec = pl.BlockSpec((AR, 128), lambda b: (0, 0))
    scal = pl.BlockSpec((1, 1), lambda b: (0, 0))
    return pl.pallas_call(
        _s2_body,
        grid=(B,),
        in_specs=[big] * 11 + [anch_spec] * 4,
        out_specs=[big, pl.BlockSpec((1, 4, AR, 128), lambda b: (b, 0, 0, 0)),
                   scal, scal, scal],
        out_shape=[jax.ShapeDtypeStruct((B, AR, 128), jnp.int32),
                   jax.ShapeDtypeStruct((B, 4, AR, 128), jnp.float32),
                   jax.ShapeDtypeStruct((1, 1), jnp.float32),
                   jax.ShapeDtypeStruct((1, 1), jnp.float32),
                   jax.ShapeDtypeStruct((1, 1), jnp.float32)],
        interpret=_INTERPRET,
    )(mi, *m, l0, l1, *p, *_AX)


def kernel(rpn_cls_logits, rpn_bbox_pred, gt_boxes, gt_labels, feat_map_shape):
    valid = gt_labels > 0
    order = jnp.argsort(jnp.where(valid, 0, 1).astype(jnp.int32),
                        axis=1, stable=True)
    cg = jnp.take_along_axis(gt_boxes, order[..., None], axis=1)
    nv = jnp.sum(valid.astype(jnp.int32), axis=1)
    nv = jnp.concatenate([nv, jnp.zeros((12,), jnp.int32)])
    ga2 = (cg[..., 2] - cg[..., 0]) * (cg[..., 3] - cg[..., 1])
    rows = jnp.stack([cg[..., 0], cg[..., 1], cg[..., 2], cg[..., 3], ga2],
                     axis=1)
    gt_pack = jnp.pad(rows, ((0, 0), (0, 0), (0, GP - NG))).reshape(-1)
    packed = _sc_match(gt_pack, nv)
    arrs = packed.reshape(NW, B, 5, APW).transpose(2, 1, 0, 3).reshape(
        5, B, AR, 128)
    mi, m0, m1, m2, m3 = arrs[0], arrs[1], arrs[2], arrs[3], arrs[4]
    l0 = rpn_cls_logits[:, :, 0].reshape(B, AR, 128)
    l1 = rpn_cls_logits[:, :, 1].reshape(B, AR, 128)
    pred = rpn_bbox_pred.reshape(B, A, 4)
    p = [pred[:, :, c].reshape(B, AR, 128) for c in range(4)]
    lab3, tgt4, ce, slv, cnt = _stage2(mi, (m0, m1, m2, m3), l0, l1, p)
    cls_loss = ce[0, 0] / float(A * B)
    bbox_loss = slv[0, 0] / jnp.maximum(cnt[0, 0], 1.0)
    labels = lab3.reshape(B, A)
    targets = jnp.transpose(tgt4.reshape(B, 4, A), (0, 2, 1))
    return cls_loss, bbox_loss, labels, targets


# speedup vs baseline: 1.0090x; 1.0090x over previous
import functools

import jax
import jax.numpy as jnp
import numpy as np
from jax import lax
from jax.experimental import pallas as pl
from jax.experimental.pallas import tpu as pltpu
from jax.experimental.pallas import tpu_sc as plsc

NA = 9
FH = 64
FW = 64
B = 4
NG = 50
A = FH * FW * NA
AR = A // 128
GP = 64
R1 = 16
NJ = AR // R1

_INTERPRET = False


def _anchors_np():
    base_size = 16
    anchors = []
    cx = base_size / 2.0
    cy = base_size / 2.0
    for r in (0.5, 1.0, 2.0):
        for s in (8, 16, 32):
            area = float(base_size * s) ** 2
            w = np.sqrt(area / r)
            h = w * r
            anchors.append([cx - 0.5 * w, cy - 0.5 * h, cx + 0.5 * w, cy + 0.5 * h])
    base = np.array(anchors, dtype=np.float32)
    shift_x = np.arange(FW, dtype=np.float32) * 16.0
    shift_y = np.arange(FH, dtype=np.float32) * 16.0
    sx, sy = np.meshgrid(shift_x, shift_y, indexing="ij")
    shifts = np.stack([sx, sy, sx, sy], axis=-1).reshape(-1, 4).astype(np.float32)
    return (base[None, :, :] + shifts[:, None, :]).reshape(-1, 4).astype(np.float32)


_ANCH = _anchors_np()
_AX = [np.ascontiguousarray(_ANCH[:, c].reshape(AR, 128)) for c in range(4)]


def _s1_body(ax1r, ay1r, ax2r, ay2r, g0r, g1r, g2r, g3r, vmr,
             mir, m0r, m1r, m2r, m3r):
    ax1 = ax1r[...][None]
    ay1 = ay1r[...][None]
    ax2 = ax2r[...][None]
    ay2 = ay2r[...][None]
    gx1 = g0r[0]
    gy1 = g1r[0]
    gx2 = g2r[0]
    gy2 = g3r[0]
    vm = vmr[0]
    x1 = jnp.maximum(ax1, gx1)
    y1 = jnp.maximum(ay1, gy1)
    x2 = jnp.minimum(ax2, gx2)
    y2 = jnp.minimum(ay2, gy2)
    inter = jnp.maximum(0.0, x2 - x1) * jnp.maximum(0.0, y2 - y1)
    a1 = (ax2 - ax1) * (ay2 - ay1)
    a2 = (gx2 - gx1) * (gy2 - gy1)
    iou = inter / (a1 + a2 - inter + 1e-8)
    iou = iou * vm + (vm - 1.0)
    mx = jnp.max(iou, axis=0)
    it = jax.lax.broadcasted_iota(jnp.int32, (GP, R1, 128), 0)
    am = jnp.min(jnp.where(iou >= mx[None], it, GP), axis=0)
    oh = (it == am[None]).astype(jnp.float32)
    mir[0] = mx
    m0r[0] = jnp.sum(oh * gx1, axis=0)
    m1r[0] = jnp.sum(oh * gy1, axis=0)
    m2r[0] = jnp.sum(oh * gx2, axis=0)
    m3r[0] = jnp.sum(oh * gy2, axis=0)


def _stage1(g, vm):
    anch_spec = pl.BlockSpec((R1, 128), lambda b, j: (j, 0))
    gt_spec = pl.BlockSpec((1, GP, 1, 1), lambda b, j: (b, 0, 0, 0))
    out_spec = pl.BlockSpec((1, R1, 128), lambda b, j: (b, j, 0))
    shp = jax.ShapeDtypeStruct((B, AR, 128), jnp.float32)
    return pl.pallas_call(
        _s1_body,
        grid=(B, NJ),
        in_specs=[anch_spec] * 4 + [gt_spec] * 5,
        out_specs=[out_spec] * 5,
        out_shape=[shp] * 5,
        interpret=_INTERPRET,
    )(*_AX, *g, vm)


NC = 1
NW = 16 * NC
APW = A // NW
NCH = APW // 16
Q = 4
_AXF = [np.ascontiguousarray(_ANCH[:, c]) for c in range(4)]


_AXPACK = np.ascontiguousarray(
    _ANCH.T.reshape(4, NW, APW).transpose(1, 0, 2).reshape(-1))


def _sc_match(gt_pack, nv):
    mesh = plsc.VectorSubcoreMesh(core_axis_name="c", subcore_axis_name="s",
                                  num_cores=NC)
    shp = jax.ShapeDtypeStruct((NW * B * 5 * APW,), jnp.float32)

    @functools.partial(
        pl.kernel,
        mesh=mesh,
        out_type=shp,
        scratch_types=[pltpu.VMEM((4 * APW,), jnp.float32),
                       pltpu.VMEM((B * 5 * GP,), jnp.float32),
                       pltpu.VMEM((16,), jnp.int32),
                       pltpu.VMEM((B * 5 * APW,), jnp.float32)],
        compiler_params=pltpu.CompilerParams(needs_layout_passes=False),
    )
    def k(axh, gth, nvh, outh, axv, gtv, nvv, outv):
        wid = lax.axis_index("s") * NC + lax.axis_index("c")
        pltpu.sync_copy(axh.at[pl.ds(wid * 4 * APW, 4 * APW)], axv)
        pltpu.sync_copy(gth, gtv)
        pltpu.sync_copy(nvh, nvv)
        for b in range(B):
            nvb = nvv[...][b]
            gof = b * 5 * GP

            def quad(q, _):
                off = [q * (16 * Q) + i * 16 for i in range(Q)]
                ax1 = [axv[pl.ds(0 * APW + o, 16)] for o in off]
                ay1 = [axv[pl.ds(1 * APW + o, 16)] for o in off]
                ax2 = [axv[pl.ds(2 * APW + o, 16)] for o in off]
                ay2 = [axv[pl.ds(3 * APW + o, 16)] for o in off]
                a1 = [(ax2[i] - ax1[i]) * (ay2[i] - ay1[i]) for i in range(Q)]

                def gt_step(g, carry):
                    rmax, ridx = carry
                    gv = jnp.full((16,), g, jnp.int32)
                    gx1 = plsc.load_gather(gtv, [gv + (gof + 0 * GP)])
                    gy1 = plsc.load_gather(gtv, [gv + (gof + 1 * GP)])
                    gx2 = plsc.load_gather(gtv, [gv + (gof + 2 * GP)])
                    gy2 = plsc.load_gather(gtv, [gv + (gof + 3 * GP)])
                    ga = plsc.load_gather(gtv, [gv + (gof + 4 * GP)])
                    nmax, nidx = [], []
                    for i in range(Q):
                        iw = jnp.maximum(
                            0.0, jnp.minimum(ax2[i], gx2) - jnp.maximum(ax1[i], gx1))
                        ih = jnp.maximum(
                            0.0, jnp.minimum(ay2[i], gy2) - jnp.maximum(ay1[i], gy1))
                        inter = iw * ih
                        iou = inter / (a1[i] + ga - inter + 1e-8)
                        upd = iou > rmax[i]
                        nmax.append(jnp.where(upd, iou, rmax[i]))
                        nidx.append(jnp.where(upd, gv, ridx[i]))
                    return tuple(nmax), tuple(nidx)

                rmax = tuple(jnp.full((16,), -1.0, jnp.float32) for _ in range(Q))
                ridx = tuple(jnp.zeros((16,), jnp.int32) for _ in range(Q))
                rmax, ridx = lax.fori_loop(0, nvb, gt_step, (rmax, ridx))
                oof = b * 5 * APW
                for i in range(Q):
                    outv[pl.ds(oof + 0 * APW + off[i], 16)] = rmax[i]
                    outv[pl.ds(oof + 1 * APW + off[i], 16)] = plsc.load_gather(
                        gtv, [ridx[i] + (gof + 0 * GP)])
                    outv[pl.ds(oof + 2 * APW + off[i], 16)] = plsc.load_gather(
                        gtv, [ridx[i] + (gof + 1 * GP)])
                    outv[pl.ds(oof + 3 * APW + off[i], 16)] = plsc.load_gather(
                        gtv, [ridx[i] + (gof + 2 * GP)])
                    outv[pl.ds(oof + 4 * APW + off[i], 16)] = plsc.load_gather(
                        gtv, [ridx[i] + (gof + 3 * GP)])
                return 0

            lax.fori_loop(0, NCH // Q, quad, 0)
        pltpu.sync_copy(outv, outh.at[pl.ds(wid * B * 5 * APW, B * 5 * APW)])

    return k(_AXPACK, gt_pack, nv)


def _smooth_l1(d):
    ad = jnp.abs(d)
    return jnp.where(ad < 1.0, 0.5 * d * d, ad - 0.5)


def _s2_body(mir, m0r, m1r, m2r, m3r, l0r, l1r, p0r, p1r, p2r, p3r,
             ax1r, ay1r, ax2r, ay2r, labr, tgtr, cer, slr, cntr):
    b = pl.program_id(0)
    mx = mir[0]
    lab = mx >= 0.7
    labf = lab.astype(jnp.float32)
    labr[0] = lab.astype(jnp.int32)
    ax1 = ax1r[...]
    ay1 = ay1r[...]
    ax2 = ax2r[...]
    ay2 = ay2r[...]
    bw = ax2 - ax1 + 1.0
    bh = ay2 - ay1 + 1.0
    bcx = ax1 + 0.5 * bw
    bcy = ay1 + 0.5 * bh
    m0 = m0r[0]
    m1 = m1r[0]
    m2 = m2r[0]
    m3 = m3r[0]
    gw = m2 - m0 + 1.0
    gh = m3 - m1 + 1.0
    gcx = m0 + 0.5 * gw
    gcy = m1 + 0.5 * gh
    t0 = (gcx - bcx) / bw
    t1 = (gcy - bcy) / bh
    t2 = jnp.log(gw / bw)
    t3 = jnp.log(gh / bh)
    tgtr[0, 0] = t0
    tgtr[0, 1] = t1
    tgtr[0, 2] = t2
    tgtr[0, 3] = t3
    l0 = l0r[0]
    l1 = l1r[0]
    mm = jnp.maximum(l0, l1)
    lse = mm + jnp.log(jnp.exp(l0 - mm) + jnp.exp(l1 - mm))
    ce_b = jnp.sum(lse - jnp.where(lab, l1, l0), keepdims=True)
    sl = (_smooth_l1(p0r[0] - t0) + _smooth_l1(p1r[0] - t1)
          + _smooth_l1(p2r[0] - t2) + _smooth_l1(p3r[0] - t3))
    sl_b = jnp.sum(sl * labf, keepdims=True)
    cnt_b = jnp.sum(labf, keepdims=True)

    @pl.when(b == 0)
    def _():
        cer[...] = jnp.zeros((1, 1), jnp.float32)
        slr[...] = jnp.zeros((1, 1), jnp.float32)
        cntr[...] = jnp.zeros((1, 1), jnp.float32)

    cer[...] += ce_b
    slr[...] += sl_b
    cntr[...] += cnt_b


def _stage2(mi, m, l0, l1, p):
    big = pl.BlockSpec((1, AR, 128), lambda b: (b, 0, 0))
    anch_spec = pl.BlockSpec((AR, 128), lambda b: (0, 0))
    scal = pl.BlockSpec((1, 1), lambda b: (0, 0))
    return pl.pallas_call(
        _s2_body,
        grid=(B,),
        in_specs=[big] * 11 + [anch_spec] * 4,
        out_specs=[big, pl.BlockSpec((1, 4, AR, 128), lambda b: (b, 0, 0, 0)),
                   scal, scal, scal],
        out_shape=[jax.ShapeDtypeStruct((B, AR, 128), jnp.int32),
                   jax.ShapeDtypeStruct((B, 4, AR, 128), jnp.float32),
                   jax.ShapeDtypeStruct((1, 1), jnp.float32),
                   jax.ShapeDtypeStruct((1, 1), jnp.float32),
                   jax.ShapeDtypeStruct((1, 1), jnp.float32)],
        interpret=_INTERPRET,
    )(mi, *m, l0, l1, *p, *_AX)


def kernel(rpn_cls_logits, rpn_bbox_pred, gt_boxes, gt_labels, feat_map_shape):
    valid = gt_labels > 0
    order = jnp.argsort(jnp.where(valid, 0, 1).astype(jnp.int32),
                        axis=1, stable=True)
    cg = jnp.take_along_axis(gt_boxes, order[..., None], axis=1)
    nv = jnp.sum(valid.astype(jnp.int32), axis=1)
    nv = jnp.concatenate([nv, jnp.zeros((12,), jnp.int32)])
    ga2 = (cg[..., 2] - cg[..., 0]) * (cg[..., 3] - cg[..., 1])
    rows = jnp.stack([cg[..., 0], cg[..., 1], cg[..., 2], cg[..., 3], ga2],
                     axis=1)
    gt_pack = jnp.pad(rows, ((0, 0), (0, 0), (0, GP - NG))).reshape(-1)
    packed = _sc_match(gt_pack, nv)
    arrs = packed.reshape(NW, B, 5, APW).transpose(2, 1, 0, 3).reshape(
        5, B, AR, 128)
    mi, m0, m1, m2, m3 = arrs[0], arrs[1], arrs[2], arrs[3], arrs[4]
    l0 = rpn_cls_logits[:, :, 0].reshape(B, AR, 128)
    l1 = rpn_cls_logits[:, :, 1].reshape(B, AR, 128)
    pred = rpn_bbox_pred.reshape(B, A, 4)
    p = [pred[:, :, c].reshape(B, AR, 128) for c in range(4)]
    lab3, tgt4, ce, slv, cnt = _stage2(mi, (m0, m1, m2, m3), l0, l1, p)
    cls_loss = ce[0, 0] / float(A * B)
    bbox_loss = slv[0, 0] / jnp.maximum(cnt[0, 0], 1.0)
    labels = lab3.reshape(B, A)
    targets = jnp.transpose(tgt4.reshape(B, 4, A), (0, 2, 1))
    return cls_loss, bbox_loss, labels, targets


# speedup vs baseline: 1.4104x; 1.3978x over previous
import functools

import jax
import jax.numpy as jnp
import numpy as np
from jax import lax
from jax.experimental import pallas as pl
from jax.experimental.pallas import tpu as pltpu
from jax.experimental.pallas import tpu_sc as plsc

NA = 9
FH = 64
FW = 64
B = 4
NG = 50
A = FH * FW * NA
AR = A // 128
GP = 64
R1 = 16
NJ = AR // R1

_INTERPRET = False


def _anchors_np():
    base_size = 16
    anchors = []
    cx = base_size / 2.0
    cy = base_size / 2.0
    for r in (0.5, 1.0, 2.0):
        for s in (8, 16, 32):
            area = float(base_size * s) ** 2
            w = np.sqrt(area / r)
            h = w * r
            anchors.append([cx - 0.5 * w, cy - 0.5 * h, cx + 0.5 * w, cy + 0.5 * h])
    base = np.array(anchors, dtype=np.float32)
    shift_x = np.arange(FW, dtype=np.float32) * 16.0
    shift_y = np.arange(FH, dtype=np.float32) * 16.0
    sx, sy = np.meshgrid(shift_x, shift_y, indexing="ij")
    shifts = np.stack([sx, sy, sx, sy], axis=-1).reshape(-1, 4).astype(np.float32)
    return (base[None, :, :] + shifts[:, None, :]).reshape(-1, 4).astype(np.float32)


_ANCH = _anchors_np()
_AX = [np.ascontiguousarray(_ANCH[:, c].reshape(AR, 128)) for c in range(4)]


def _s1_body(ax1r, ay1r, ax2r, ay2r, g0r, g1r, g2r, g3r, vmr,
             mir, m0r, m1r, m2r, m3r):
    ax1 = ax1r[...][None]
    ay1 = ay1r[...][None]
    ax2 = ax2r[...][None]
    ay2 = ay2r[...][None]
    gx1 = g0r[0]
    gy1 = g1r[0]
    gx2 = g2r[0]
    gy2 = g3r[0]
    vm = vmr[0]
    x1 = jnp.maximum(ax1, gx1)
    y1 = jnp.maximum(ay1, gy1)
    x2 = jnp.minimum(ax2, gx2)
    y2 = jnp.minimum(ay2, gy2)
    inter = jnp.maximum(0.0, x2 - x1) * jnp.maximum(0.0, y2 - y1)
    a1 = (ax2 - ax1) * (ay2 - ay1)
    a2 = (gx2 - gx1) * (gy2 - gy1)
    iou = inter / (a1 + a2 - inter + 1e-8)
    iou = iou * vm + (vm - 1.0)
    mx = jnp.max(iou, axis=0)
    it = jax.lax.broadcasted_iota(jnp.int32, (GP, R1, 128), 0)
    am = jnp.min(jnp.where(iou >= mx[None], it, GP), axis=0)
    oh = (it == am[None]).astype(jnp.float32)
    mir[0] = mx
    m0r[0] = jnp.sum(oh * gx1, axis=0)
    m1r[0] = jnp.sum(oh * gy1, axis=0)
    m2r[0] = jnp.sum(oh * gx2, axis=0)
    m3r[0] = jnp.sum(oh * gy2, axis=0)


def _stage1(g, vm):
    anch_spec = pl.BlockSpec((R1, 128), lambda b, j: (j, 0))
    gt_spec = pl.BlockSpec((1, GP, 1, 1), lambda b, j: (b, 0, 0, 0))
    out_spec = pl.BlockSpec((1, R1, 128), lambda b, j: (b, j, 0))
    shp = jax.ShapeDtypeStruct((B, AR, 128), jnp.float32)
    return pl.pallas_call(
        _s1_body,
        grid=(B, NJ),
        in_specs=[anch_spec] * 4 + [gt_spec] * 5,
        out_specs=[out_spec] * 5,
        out_shape=[shp] * 5,
        interpret=_INTERPRET,
    )(*_AX, *g, vm)


NC = 1
NW = 16 * NC
APW = A // NW
NCH = APW // 16
Q = 4
_AXF = [np.ascontiguousarray(_ANCH[:, c]) for c in range(4)]


_AXPACK = np.ascontiguousarray(
    _ANCH.T.reshape(4, NW, APW).transpose(1, 0, 2).reshape(-1))


def _sc_match(gt_pack):
    mesh = plsc.VectorSubcoreMesh(core_axis_name="c", subcore_axis_name="s",
                                  num_cores=NC)
    shp = jax.ShapeDtypeStruct((NW * B * 5 * APW,), jnp.float32)

    @functools.partial(
        pl.kernel,
        mesh=mesh,
        out_type=shp,
        scratch_types=[pltpu.VMEM((4 * APW,), jnp.float32),
                       pltpu.VMEM((B * 6 * GP,), jnp.float32),
                       pltpu.VMEM((B * 5 * APW,), jnp.float32)],
        compiler_params=pltpu.CompilerParams(needs_layout_passes=False),
    )
    def k(axh, gth, outh, axv, gtv, outv):
        wid = lax.axis_index("s") * NC + lax.axis_index("c")
        pltpu.sync_copy(axh.at[pl.ds(wid * 4 * APW, 4 * APW)], axv)
        pltpu.sync_copy(gth, gtv)
        for b in range(B):
            gof = b * 6 * GP

            def quad(q, _):
                off = [q * (16 * Q) + i * 16 for i in range(Q)]
                ax1 = [axv[pl.ds(0 * APW + o, 16)] for o in off]
                ay1 = [axv[pl.ds(1 * APW + o, 16)] for o in off]
                ax2 = [axv[pl.ds(2 * APW + o, 16)] for o in off]
                ay2 = [axv[pl.ds(3 * APW + o, 16)] for o in off]
                a1 = [(ax2[i] - ax1[i]) * (ay2[i] - ay1[i]) for i in range(Q)]

                def gt_step(g, carry):
                    rmax, ridx = carry
                    gv = jnp.full((16,), g, jnp.int32)
                    gx1 = plsc.load_gather(gtv, [gv + (gof + 0 * GP)])
                    gy1 = plsc.load_gather(gtv, [gv + (gof + 1 * GP)])
                    gx2 = plsc.load_gather(gtv, [gv + (gof + 2 * GP)])
                    gy2 = plsc.load_gather(gtv, [gv + (gof + 3 * GP)])
                    ga = plsc.load_gather(gtv, [gv + (gof + 4 * GP)])
                    vmm1 = plsc.load_gather(gtv, [gv + (gof + 5 * GP)])
                    vm = vmm1 + 1.0
                    nmax, nidx = [], []
                    for i in range(Q):
                        iw = jnp.maximum(
                            0.0, jnp.minimum(ax2[i], gx2) - jnp.maximum(ax1[i], gx1))
                        ih = jnp.maximum(
                            0.0, jnp.minimum(ay2[i], gy2) - jnp.maximum(ay1[i], gy1))
                        inter = iw * ih
                        iou = inter / (a1[i] + ga - inter + 1e-8)
                        iou = iou * vm + vmm1
                        upd = iou > rmax[i]
                        nmax.append(jnp.where(upd, iou, rmax[i]))
                        nidx.append(jnp.where(upd, gv, ridx[i]))
                    return tuple(nmax), tuple(nidx)

                rmax = tuple(jnp.full((16,), -2.0, jnp.float32) for _ in range(Q))
                ridx = tuple(jnp.zeros((16,), jnp.int32) for _ in range(Q))
                rmax, ridx = lax.fori_loop(0, NG, gt_step, (rmax, ridx))
                oof = b * 5 * APW
                for i in range(Q):
                    outv[pl.ds(oof + 0 * APW + off[i], 16)] = rmax[i]
                    outv[pl.ds(oof + 1 * APW + off[i], 16)] = plsc.load_gather(
                        gtv, [ridx[i] + (gof + 0 * GP)])
                    outv[pl.ds(oof + 2 * APW + off[i], 16)] = plsc.load_gather(
                        gtv, [ridx[i] + (gof + 1 * GP)])
                    outv[pl.ds(oof + 3 * APW + off[i], 16)] = plsc.load_gather(
                        gtv, [ridx[i] + (gof + 2 * GP)])
                    outv[pl.ds(oof + 4 * APW + off[i], 16)] = plsc.load_gather(
                        gtv, [ridx[i] + (gof + 3 * GP)])
                return 0

            lax.fori_loop(0, NCH // Q, quad, 0)
        pltpu.sync_copy(outv, outh.at[pl.ds(wid * B * 5 * APW, B * 5 * APW)])

    return k(_AXPACK, gt_pack)


def _smooth_l1(d):
    ad = jnp.abs(d)
    return jnp.where(ad < 1.0, 0.5 * d * d, ad - 0.5)


def _s2_body(mir, m0r, m1r, m2r, m3r, ltr, ptr,
             ax1r, ay1r, ax2r, ay2r, labr, tgtr, cer, slr, cntr):
    b = pl.program_id(0)
    mx = mir[0]
    lab = mx >= 0.7
    labf = lab.astype(jnp.float32)
    labr[0] = lab.astype(jnp.int32)
    ax1 = ax1r[...]
    ay1 = ay1r[...]
    ax2 = ax2r[...]
    ay2 = ay2r[...]
    bw = ax2 - ax1 + 1.0
    bh = ay2 - ay1 + 1.0
    bcx = ax1 + 0.5 * bw
    bcy = ay1 + 0.5 * bh
    m0 = m0r[0]
    m1 = m1r[0]
    m2 = m2r[0]
    m3 = m3r[0]
    gw = m2 - m0 + 1.0
    gh = m3 - m1 + 1.0
    gcx = m0 + 0.5 * gw
    gcy = m1 + 0.5 * gh
    t0 = (gcx - bcx) / bw
    t1 = (gcy - bcy) / bh
    t2 = jnp.log(gw / bw)
    t3 = jnp.log(gh / bh)
    tgtr[0, 0] = t0
    tgtr[0, 1] = t1
    tgtr[0, 2] = t2
    tgtr[0, 3] = t3
    l0 = ltr[0, 0]
    l1 = ltr[0, 1]
    mm = jnp.maximum(l0, l1)
    lse = mm + jnp.log(jnp.exp(l0 - mm) + jnp.exp(l1 - mm))
    ce_b = jnp.sum(lse - jnp.where(lab, l1, l0), keepdims=True)
    sl = (_smooth_l1(ptr[0, 0] - t0) + _smooth_l1(ptr[0, 1] - t1)
          + _smooth_l1(ptr[0, 2] - t2) + _smooth_l1(ptr[0, 3] - t3))
    sl_b = jnp.sum(sl * labf, keepdims=True)
    cnt_b = jnp.sum(labf, keepdims=True)

    @pl.when(b == 0)
    def _():
        cer[...] = jnp.zeros((1, 1), jnp.float32)
        slr[...] = jnp.zeros((1, 1), jnp.float32)
        cntr[...] = jnp.zeros((1, 1), jnp.float32)

    cer[...] += ce_b
    slr[...] += sl_b
    cntr[...] += cnt_b


def _stage2(mi, m, lt, pt):
    big = pl.BlockSpec((1, AR, 128), lambda b: (b, 0, 0))
    anch_spec = pl.BlockSpec((AR, 128), lambda b: (0, 0))
    scal = pl.BlockSpec((1, 1), lambda b: (0, 0))
    return pl.pallas_call(
        _s2_body,
        grid=(B,),
        in_specs=[big] * 5
        + [pl.BlockSpec((1, 2, AR, 128), lambda b: (b, 0, 0, 0)),
           pl.BlockSpec((1, 4, AR, 128), lambda b: (b, 0, 0, 0))]
        + [anch_spec] * 4,
        out_specs=[big, pl.BlockSpec((1, 4, AR, 128), lambda b: (b, 0, 0, 0)),
                   scal, scal, scal],
        out_shape=[jax.ShapeDtypeStruct((B, AR, 128), jnp.int32),
                   jax.ShapeDtypeStruct((B, 4, AR, 128), jnp.float32),
                   jax.ShapeDtypeStruct((1, 1), jnp.float32),
                   jax.ShapeDtypeStruct((1, 1), jnp.float32),
                   jax.ShapeDtypeStruct((1, 1), jnp.float32)],
        interpret=_INTERPRET,
    )(mi, *m, lt, pt, *_AX)


def kernel(rpn_cls_logits, rpn_bbox_pred, gt_boxes, gt_labels, feat_map_shape):
    ga2 = ((gt_boxes[..., 2] - gt_boxes[..., 0])
           * (gt_boxes[..., 3] - gt_boxes[..., 1]))
    vmm1 = (gt_labels > 0).astype(jnp.float32) - 1.0
    rows = jnp.stack([gt_boxes[..., 0], gt_boxes[..., 1], gt_boxes[..., 2],
                      gt_boxes[..., 3], ga2, vmm1], axis=1)
    gt_pack = jnp.pad(rows, ((0, 0), (0, 0), (0, GP - NG)),
                      constant_values=-1.0).reshape(-1)
    packed = _sc_match(gt_pack)
    arrs = packed.reshape(NW, B, 5, APW).transpose(2, 1, 0, 3).reshape(
        5, B, AR, 128)
    mi, m0, m1, m2, m3 = arrs[0], arrs[1], arrs[2], arrs[3], arrs[4]
    lt = jnp.transpose(rpn_cls_logits, (0, 2, 1)).reshape(B, 2, AR, 128)
    pt = jnp.transpose(rpn_bbox_pred.reshape(B, A, 4),
                       (0, 2, 1)).reshape(B, 4, AR, 128)
    lab3, tgt4, ce, slv, cnt = _stage2(mi, (m0, m1, m2, m3), lt, pt)
    cls_loss = ce[0, 0] / float(A * B)
    bbox_loss = slv[0, 0] / jnp.maximum(cnt[0, 0], 1.0)
    labels = lab3.reshape(B, A)
    targets = jnp.transpose(tgt4.reshape(B, 4, A), (0, 2, 1))
    return cls_loss, bbox_loss, labels, targets


# speedup vs baseline: 1.4108x; 1.0003x over previous
import functools

import jax
import jax.numpy as jnp
import numpy as np
from jax import lax
from jax.experimental import pallas as pl
from jax.experimental.pallas import tpu as pltpu
from jax.experimental.pallas import tpu_sc as plsc

NA = 9
FH = 64
FW = 64
B = 4
NG = 50
A = FH * FW * NA
AR = A // 128
GP = 64

def _anchors_np():
    base_size = 16
    anchors = []
    cx = base_size / 2.0
    cy = base_size / 2.0
    for r in (0.5, 1.0, 2.0):
        for s in (8, 16, 32):
            area = float(base_size * s) ** 2
            w = np.sqrt(area / r)
            h = w * r
            anchors.append([cx - 0.5 * w, cy - 0.5 * h, cx + 0.5 * w, cy + 0.5 * h])
    base = np.array(anchors, dtype=np.float32)
    shift_x = np.arange(FW, dtype=np.float32) * 16.0
    shift_y = np.arange(FH, dtype=np.float32) * 16.0
    sx, sy = np.meshgrid(shift_x, shift_y, indexing="ij")
    shifts = np.stack([sx, sy, sx, sy], axis=-1).reshape(-1, 4).astype(np.float32)
    return (base[None, :, :] + shifts[:, None, :]).reshape(-1, 4).astype(np.float32)


_ANCH = _anchors_np()
_AX = [np.ascontiguousarray(_ANCH[:, c].reshape(AR, 128)) for c in range(4)]


NC = 1
NW = 16 * NC
APW = A // NW
NCH = APW // 16
Q = 4
_AXPACK = np.ascontiguousarray(
    _ANCH.T.reshape(4, NW, APW).transpose(1, 0, 2).reshape(-1))


def _sc_match(gt_pack):
    mesh = plsc.VectorSubcoreMesh(core_axis_name="c", subcore_axis_name="s",
                                  num_cores=NC)
    shp = jax.ShapeDtypeStruct((NW * B * 5 * APW,), jnp.float32)

    @functools.partial(
        pl.kernel,
        mesh=mesh,
        out_type=shp,
        scratch_types=[pltpu.VMEM((4 * APW,), jnp.float32),
                       pltpu.VMEM((B * 6 * GP,), jnp.float32),
                       pltpu.VMEM((B * 5 * APW,), jnp.float32)],
        compiler_params=pltpu.CompilerParams(needs_layout_passes=False),
    )
    def k(axh, gth, outh, axv, gtv, outv):
        wid = lax.axis_index("s") * NC + lax.axis_index("c")
        pltpu.sync_copy(axh.at[pl.ds(wid * 4 * APW, 4 * APW)], axv)
        pltpu.sync_copy(gth, gtv)
        for b in range(B):
            gof = b * 6 * GP

            def quad(q, _):
                off = [q * (16 * Q) + i * 16 for i in range(Q)]
                ax1 = [axv[pl.ds(0 * APW + o, 16)] for o in off]
                ay1 = [axv[pl.ds(1 * APW + o, 16)] for o in off]
                ax2 = [axv[pl.ds(2 * APW + o, 16)] for o in off]
                ay2 = [axv[pl.ds(3 * APW + o, 16)] for o in off]
                a1 = [(ax2[i] - ax1[i]) * (ay2[i] - ay1[i]) for i in range(Q)]

                def gt_step(g, carry):
                    rmax, ridx = carry
                    gv = jnp.full((16,), g, jnp.int32)
                    gx1 = plsc.load_gather(gtv, [gv + (gof + 0 * GP)])
                    gy1 = plsc.load_gather(gtv, [gv + (gof + 1 * GP)])
                    gx2 = plsc.load_gather(gtv, [gv + (gof + 2 * GP)])
                    gy2 = plsc.load_gather(gtv, [gv + (gof + 3 * GP)])
                    ga = plsc.load_gather(gtv, [gv + (gof + 4 * GP)])
                    vmm1 = plsc.load_gather(gtv, [gv + (gof + 5 * GP)])
                    vm = vmm1 + 1.0
                    nmax, nidx = [], []
                    for i in range(Q):
                        iw = jnp.maximum(
                            0.0, jnp.minimum(ax2[i], gx2) - jnp.maximum(ax1[i], gx1))
                        ih = jnp.maximum(
                            0.0, jnp.minimum(ay2[i], gy2) - jnp.maximum(ay1[i], gy1))
                        inter = iw * ih
                        iou = inter / (a1[i] + ga - inter + 1e-8)
                        iou = iou * vm + vmm1
                        upd = iou > rmax[i]
                        nmax.append(jnp.where(upd, iou, rmax[i]))
                        nidx.append(jnp.where(upd, gv, ridx[i]))
                    return tuple(nmax), tuple(nidx)

                rmax = tuple(jnp.full((16,), -2.0, jnp.float32) for _ in range(Q))
                ridx = tuple(jnp.zeros((16,), jnp.int32) for _ in range(Q))
                rmax, ridx = lax.fori_loop(0, NG, gt_step, (rmax, ridx))
                oof = b * 5 * APW
                for i in range(Q):
                    outv[pl.ds(oof + 0 * APW + off[i], 16)] = rmax[i]
                    outv[pl.ds(oof + 1 * APW + off[i], 16)] = plsc.load_gather(
                        gtv, [ridx[i] + (gof + 0 * GP)])
                    outv[pl.ds(oof + 2 * APW + off[i], 16)] = plsc.load_gather(
                        gtv, [ridx[i] + (gof + 1 * GP)])
                    outv[pl.ds(oof + 3 * APW + off[i], 16)] = plsc.load_gather(
                        gtv, [ridx[i] + (gof + 2 * GP)])
                    outv[pl.ds(oof + 4 * APW + off[i], 16)] = plsc.load_gather(
                        gtv, [ridx[i] + (gof + 3 * GP)])
                return 0

            lax.fori_loop(0, NCH // Q, quad, 0)
        pltpu.sync_copy(outv, outh.at[pl.ds(wid * B * 5 * APW, B * 5 * APW)])

    return k(_AXPACK, gt_pack)


def _smooth_l1(d):
    ad = jnp.abs(d)
    return jnp.where(ad < 1.0, 0.5 * d * d, ad - 0.5)


def _s2_body(mir, m0r, m1r, m2r, m3r, ltr, ptr,
             ax1r, ay1r, ax2r, ay2r, labr, tgtr, cer, slr, cntr):
    b = pl.program_id(0)
    mx = mir[0]
    lab = mx >= 0.7
    labf = lab.astype(jnp.float32)
    labr[0] = lab.astype(jnp.int32)
    ax1 = ax1r[...]
    ay1 = ay1r[...]
    ax2 = ax2r[...]
    ay2 = ay2r[...]
    bw = ax2 - ax1 + 1.0
    bh = ay2 - ay1 + 1.0
    bcx = ax1 + 0.5 * bw
    bcy = ay1 + 0.5 * bh
    m0 = m0r[0]
    m1 = m1r[0]
    m2 = m2r[0]
    m3 = m3r[0]
    gw = m2 - m0 + 1.0
    gh = m3 - m1 + 1.0
    gcx = m0 + 0.5 * gw
    gcy = m1 + 0.5 * gh
    t0 = (gcx - bcx) / bw
    t1 = (gcy - bcy) / bh
    t2 = jnp.log(gw / bw)
    t3 = jnp.log(gh / bh)
    tgtr[0, 0] = t0
    tgtr[0, 1] = t1
    tgtr[0, 2] = t2
    tgtr[0, 3] = t3
    l0 = ltr[0, 0]
    l1 = ltr[0, 1]
    mm = jnp.maximum(l0, l1)
    lse = mm + jnp.log(jnp.exp(l0 - mm) + jnp.exp(l1 - mm))
    ce_b = jnp.sum(lse - jnp.where(lab, l1, l0), keepdims=True)
    sl = (_smooth_l1(ptr[0, 0] - t0) + _smooth_l1(ptr[0, 1] - t1)
          + _smooth_l1(ptr[0, 2] - t2) + _smooth_l1(ptr[0, 3] - t3))
    sl_b = jnp.sum(sl * labf, keepdims=True)
    cnt_b = jnp.sum(labf, keepdims=True)

    @pl.when(b == 0)
    def _():
        cer[...] = jnp.zeros((1, 1), jnp.float32)
        slr[...] = jnp.zeros((1, 1), jnp.float32)
        cntr[...] = jnp.zeros((1, 1), jnp.float32)

    cer[...] += ce_b
    slr[...] += sl_b
    cntr[...] += cnt_b


def _stage2(mi, m, lt, pt):
    big = pl.BlockSpec((1, AR, 128), lambda b: (b, 0, 0))
    anch_spec = pl.BlockSpec((AR, 128), lambda b: (0, 0))
    scal = pl.BlockSpec((1, 1), lambda b: (0, 0))
    return pl.pallas_call(
        _s2_body,
        grid=(B,),
        in_specs=[big] * 5
        + [pl.BlockSpec((1, 2, AR, 128), lambda b: (b, 0, 0, 0)),
           pl.BlockSpec((1, 4, AR, 128), lambda b: (b, 0, 0, 0))]
        + [anch_spec] * 4,
        out_specs=[big, pl.BlockSpec((1, 4, AR, 128), lambda b: (b, 0, 0, 0)),
                   scal, scal, scal],
        out_shape=[jax.ShapeDtypeStruct((B, AR, 128), jnp.int32),
                   jax.ShapeDtypeStruct((B, 4, AR, 128), jnp.float32),
                   jax.ShapeDtypeStruct((1, 1), jnp.float32),
                   jax.ShapeDtypeStruct((1, 1), jnp.float32),
                   jax.ShapeDtypeStruct((1, 1), jnp.float32)],
    )(mi, *m, lt, pt, *_AX)


def kernel(rpn_cls_logits, rpn_bbox_pred, gt_boxes, gt_labels, feat_map_shape):
    ga2 = ((gt_boxes[..., 2] - gt_boxes[..., 0])
           * (gt_boxes[..., 3] - gt_boxes[..., 1]))
    vmm1 = (gt_labels > 0).astype(jnp.float32) - 1.0
    rows = jnp.stack([gt_boxes[..., 0], gt_boxes[..., 1], gt_boxes[..., 2],
                      gt_boxes[..., 3], ga2, vmm1], axis=1)
    gt_pack = jnp.pad(rows, ((0, 0), (0, 0), (0, GP - NG)),
                      constant_values=-1.0).reshape(-1)
    packed = _sc_match(gt_pack)
    arrs = packed.reshape(NW, B, 5, APW).transpose(2, 1, 0, 3).reshape(
        5, B, AR, 128)
    mi, m0, m1, m2, m3 = arrs[0], arrs[1], arrs[2], arrs[3], arrs[4]
    lt = jnp.transpose(rpn_cls_logits, (0, 2, 1)).reshape(B, 2, AR, 128)
    pt = jnp.transpose(rpn_bbox_pred.reshape(B, A, 4),
                       (0, 2, 1)).reshape(B, 4, AR, 128)
    lab3, tgt4, ce, slv, cnt = _stage2(mi, (m0, m1, m2, m3), lt, pt)
    cls_loss = ce[0, 0] / float(A * B)
    bbox_loss = slv[0, 0] / jnp.maximum(cnt[0, 0], 1.0)
    labels = lab3.reshape(B, A)
    targets = jnp.transpose(tgt4.reshape(B, 4, A), (0, 2, 1))
    return cls_loss, bbox_loss, labels, targets
